# baseline (speedup 1.0000x reference)
import functools

import jax
import jax.numpy as jnp
from jax import lax
from jax.experimental import pallas as pl
from jax.experimental.pallas import tpu as pltpu
from jax.experimental.pallas import tpu_sc as plsc

_L = 16
_U1 = 1025
_NVREG = 64


def _shuf(x, idx):
    return x.at[idx].get(mode="promise_in_bounds", unique_indices=True)


def _scan(buf, pen, tailb, vl, vglob, lane):
    bounds = [0, 16, 32, 48, 64]
    accs = []
    for a in range(4):
        rmax = jnp.full((_L,), -3.0, jnp.float32)
        ridx = jnp.zeros((_L,), jnp.int32)
        for t in range(bounds[a] // 2, bounds[a + 1] // 2):
            pvi = pen[pl.ds(t * _L, _L)]
            pab = plsc.unpack(plsc.bitcast(pvi, jnp.bfloat16),
                              format=plsc.PackFormat.INTERLEAVED)
            for h in (0, 1):
                j = 2 * t + h
                xv = buf[vl, pl.ds(j * _L, _L)]
                eff = xv + pab[h]
                pred = eff > rmax
                rmax = jnp.maximum(eff, rmax)
                ridx = jnp.where(pred, jnp.full((_L,), j, jnp.int32), ridx)
        accs.append((rmax, ridx))
    while len(accs) > 1:
        merged = []
        for (ra, ia), (rb, ib) in zip(accs[0::2], accs[1::2]):
            pred = rb > ra
            merged.append((jnp.maximum(ra, rb), jnp.where(pred, ib, ia)))
        accs = merged
    rmax, ridx = accs[0]
    gidx = ridx * _L + lane

    tvec = tailb[pl.ds(pl.multiple_of((vglob >> 4) * _L, _L), _L)]
    tval = _shuf(tvec, jnp.full((_L,), vglob & 15, jnp.int32))
    pv2t = pen[pl.ds((_NVREG // 2) * _L, _L)]
    pt, _unused = plsc.unpack(plsc.bitcast(pv2t, jnp.bfloat16),
                              format=plsc.PackFormat.INTERLEAVED)
    efft = tval + pt
    efft = jnp.where(lane == 0, efft, jnp.float32(-1.0))
    predt = efft > rmax
    rmax = jnp.maximum(efft, rmax)
    gidx = jnp.where(predt, lane + _NVREG * _L, gidx)
    return rmax, gidx


def _pick(rmax, gidx, pen, lane):
    maxv = jnp.max(rmax)
    cand = jnp.where(rmax == maxv, gidx, jnp.int32(1 << 30))
    sel_s = jnp.min(cand)
    sel = jnp.full((_L,), sel_s, jnp.int32)

    jsel = sel_s >> 4
    lsel = sel_s & 15
    tsel = jsel >> 1
    half = jsel & 1
    gate = lax.select(sel_s > 0, jnp.int32(0xC080), jnp.int32(0))
    maski = gate << (half * 16)
    poff = pl.multiple_of(tsel * _L, _L)
    pi = pen[pl.ds(poff, _L)]
    pen[pl.ds(poff, _L)] = jnp.where(lane == lsel, pi | maski, pi)
    return maxv, sel


def _greedy_body(nrows, nchunks, vb, xr, tl, outsz, outseq,
                 bufa0, bufa1, bufb0, bufb1, pena, penb, tailba, tailbb,
                 seqba, seqbb, szb, sema0, sema1, semb0, semb1):
    c = lax.axis_index("c")
    s = lax.axis_index("s")
    nc = plsc.get_sparse_core_info().num_cores
    wid = s * nc + c
    lane = lax.iota(jnp.int32, _L)
    nv = nchunks * vb

    szb[...] = jnp.zeros((_L,), jnp.float32)

    def row_body(r, _):
        ba = wid * nrows + r
        bb = wid * nrows + nrows // 2 + r

        def pen_init(t, _):
            off = pl.ds(pl.multiple_of(t * _L, _L), _L)
            pena[off] = jnp.zeros((_L,), jnp.int32)
            penb[off] = jnp.zeros((_L,), jnp.int32)
            return 0
        lax.fori_loop(0, _NVREG // 2 + 1, pen_init, 0)

        pltpu.sync_copy(tl.at[pl.ds(ba * nv, nv)], tailba)
        pltpu.sync_copy(tl.at[pl.ds(bb * nv, nv)], tailbb)
        cpa = [pltpu.async_copy(xr.at[ba, pl.ds(0, vb), :], bufa0, sema0)]
        cpb = [pltpu.async_copy(xr.at[bb, pl.ds(0, vb), :], bufb0, semb0)]

        def v_step(bfa, bfb, ci, vl, carry):
            sza, sqa, szbv, sqb = carry
            vglob = ci * vb + vl
            ra, ga = _scan(bfa, pena, tailba, vl, vglob, lane)
            rb, gb = _scan(bfb, penb, tailbb, vl, vglob, lane)
            maxva, sela = _pick(ra, ga, pena, lane)
            maxvb, selb = _pick(rb, gb, penb, lane)

            sqa = jnp.where(lane == (vglob & 15), sela, sqa)
            sqb = jnp.where(lane == (vglob & 15), selb, sqb)

            @pl.when((vglob & 15) == 15)
            def _():
                qoff = pl.ds(pl.multiple_of((vglob >> 4) * _L, _L), _L)
                seqba[qoff] = sqa
                seqbb[qoff] = sqb

            return (sza + maxva, sqa, szbv + maxvb, sqb)

        carry = (jnp.zeros((_L,), jnp.float32), jnp.zeros((_L,), jnp.int32),
                 jnp.zeros((_L,), jnp.float32), jnp.zeros((_L,), jnp.int32))
        for ci in range(nchunks):
            bfa = bufa0 if ci % 2 == 0 else bufa1
            bfb = bufb0 if ci % 2 == 0 else bufb1
            if ci + 1 < nchunks:
                na, sa = (bufa0, sema0) if (ci + 1) % 2 == 0 else (bufa1, sema1)
                nb, sb = (bufb0, semb0) if (ci + 1) % 2 == 0 else (bufb1, semb1)
                cpa.append(pltpu.async_copy(
                    xr.at[ba, pl.ds((ci + 1) * vb, vb), :], na, sa))
                cpb.append(pltpu.async_copy(
                    xr.at[bb, pl.ds((ci + 1) * vb, vb), :], nb, sb))
            cpa[ci].wait()
            cpb[ci].wait()
            carry = lax.fori_loop(
                0, vb, functools.partial(v_step, bfa, bfb, ci), carry)
        sza, _, szbv, _ = carry

        szb[...] = jnp.where(lane == r, -sza, szb[...])
        szb[...] = jnp.where(lane == r + nrows // 2, -szbv, szb[...])
        pltpu.sync_copy(seqba, outseq.at[pl.ds(ba * nv, nv)])
        pltpu.sync_copy(seqbb, outseq.at[pl.ds(bb * nv, nv)])
        return 0

    lax.fori_loop(0, nrows // 2, row_body, 0)
    pltpu.sync_copy(szb, outsz.at[pl.ds(wid * _L, _L)])


def _tc_body(xref, szref, seqref):
    gr, V, U1 = 8, seqref.shape[1], xref.shape[2]
    iot = lax.broadcasted_iota(jnp.int32, (gr, U1), 1)
    iotv = lax.broadcasted_iota(jnp.int32, (gr, V), 1)

    def step(v, carry):
        mask, size, seq = carry
        w = xref[:, v, :]
        eff = jnp.where(mask > 0.5, jnp.float32(-1.0), w)
        m = jnp.max(eff, axis=1, keepdims=True)
        cand = jnp.where(eff == m, iot, jnp.int32(1 << 30))
        sel = jnp.min(cand, axis=1, keepdims=True)
        size = size + m
        upd = (iot == sel) & (sel > 0)
        mask = jnp.where(upd, jnp.float32(1.0), mask)
        seq = jnp.where(iotv == v, sel, seq)
        return (mask, size, seq)

    mask0 = jnp.zeros((gr, U1), jnp.float32)
    size0 = jnp.zeros((gr, 1), jnp.float32)
    seq0 = jnp.zeros((gr, V), jnp.int32)
    _, size, seq = lax.fori_loop(0, V, step, (mask0, size0, seq0))
    szref[...] = jnp.where(
        lax.broadcasted_iota(jnp.int32, (gr, 128), 1) == 0, -size,
        jnp.float32(0.0))
    seqref[...] = seq


def kernel(x, u_size, v_size):
    B, V, U1 = x.shape
    info = plsc.get_sparse_core_info()
    nw = info.num_cores * info.num_subcores
    half = B // 2
    nrows = half // nw
    vb = 16
    nchunks = V // vb

    tl = x[:, :, U1 - 1].reshape(B * V)

    gr = 8
    ng = half // gr
    tc_run = pl.pallas_call(
        _tc_body,
        grid=(ng,),
        in_specs=[pl.BlockSpec((gr, V, U1), lambda g: (g + ng, 0, 0))],
        out_specs=[
            pl.BlockSpec((gr, 128), lambda g: (g, 0)),
            pl.BlockSpec((gr, V), lambda g: (g, 0)),
        ],
        out_shape=[
            jax.ShapeDtypeStruct((half, 128), jnp.float32),
            jax.ShapeDtypeStruct((half, V), jnp.int32),
        ],
    )

    body = functools.partial(_greedy_body, nrows, nchunks, vb)
    run = pl.kernel(
        body,
        mesh=plsc.VectorSubcoreMesh(core_axis_name="c", subcore_axis_name="s"),
        out_type=[
            jax.ShapeDtypeStruct((nw * _L,), jnp.float32),
            jax.ShapeDtypeStruct((half * V,), jnp.int32),
        ],
        scratch_types=[
            pltpu.VMEM((vb, _U1), jnp.float32),
            pltpu.VMEM((vb, _U1), jnp.float32),
            pltpu.VMEM((vb, _U1), jnp.float32),
            pltpu.VMEM((vb, _U1), jnp.float32),
            pltpu.VMEM(((_NVREG // 2 + 1) * _L,), jnp.int32),
            pltpu.VMEM(((_NVREG // 2 + 1) * _L,), jnp.int32),
            pltpu.VMEM((V,), jnp.float32),
            pltpu.VMEM((V,), jnp.float32),
            pltpu.VMEM((V,), jnp.int32),
            pltpu.VMEM((V,), jnp.int32),
            pltpu.VMEM((_L,), jnp.float32),
            pltpu.SemaphoreType.DMA,
            pltpu.SemaphoreType.DMA,
            pltpu.SemaphoreType.DMA,
            pltpu.SemaphoreType.DMA,
        ],
        compiler_params=pltpu.CompilerParams(
            use_tc_tiling_on_sc=True, needs_layout_passes=False),
    )
    outsz, outseq = run(x, tl)
    tsz, tseq = tc_run(x)
    neg_size = jnp.concatenate(
        [outsz.reshape(nw, _L)[:, :nrows].reshape(half), tsz[:, 0]])
    seq = jnp.concatenate([outseq.reshape(half, V), tseq])
    return (neg_size, seq)


# speedup vs baseline: 2.8396x; 2.8396x over previous
import functools

import jax
import jax.numpy as jnp
from jax import lax
from jax.experimental import pallas as pl
from jax.experimental.pallas import tpu as pltpu
from jax.experimental.pallas import tpu_sc as plsc

_L = 16
_U1 = 1025
_NVREG = 64


def _shuf(x, idx):
    return x.at[idx].get(mode="promise_in_bounds", unique_indices=True)


def _scan(buf, pen, tailb, vl, vglob, lane):
    bounds = [0, 16, 32, 48, 64]
    accs = []
    for a in range(4):
        rmax = jnp.full((_L,), -3.0, jnp.float32)
        ridx = jnp.zeros((_L,), jnp.int32)
        for t in range(bounds[a] // 2, bounds[a + 1] // 2):
            pvi = pen[pl.ds(t * _L, _L)]
            pab = plsc.unpack(plsc.bitcast(pvi, jnp.bfloat16),
                              format=plsc.PackFormat.INTERLEAVED)
            for h in (0, 1):
                j = 2 * t + h
                xv = buf[vl, pl.ds(j * _L, _L)]
                eff = xv + pab[h]
                pred = eff > rmax
                rmax = jnp.maximum(eff, rmax)
                ridx = jnp.where(pred, jnp.full((_L,), j, jnp.int32), ridx)
        accs.append((rmax, ridx))
    while len(accs) > 1:
        merged = []
        for (ra, ia), (rb, ib) in zip(accs[0::2], accs[1::2]):
            pred = rb > ra
            merged.append((jnp.maximum(ra, rb), jnp.where(pred, ib, ia)))
        accs = merged
    rmax, ridx = accs[0]
    gidx = ridx * _L + lane

    tvec = tailb[pl.ds(pl.multiple_of((vglob >> 4) * _L, _L), _L)]
    tval = _shuf(tvec, jnp.full((_L,), vglob & 15, jnp.int32))
    pv2t = pen[pl.ds((_NVREG // 2) * _L, _L)]
    pt, _unused = plsc.unpack(plsc.bitcast(pv2t, jnp.bfloat16),
                              format=plsc.PackFormat.INTERLEAVED)
    efft = tval + pt
    efft = jnp.where(lane == 0, efft, jnp.float32(-1.0))
    predt = efft > rmax
    rmax = jnp.maximum(efft, rmax)
    gidx = jnp.where(predt, lane + _NVREG * _L, gidx)
    return rmax, gidx


def _pick(rmax, gidx, pen, lane):
    maxv = jnp.max(rmax)
    cand = jnp.where(rmax == maxv, gidx, jnp.int32(1 << 30))
    sel_s = jnp.min(cand)
    sel = jnp.full((_L,), sel_s, jnp.int32)

    jsel = sel_s >> 4
    lsel = sel_s & 15
    tsel = jsel >> 1
    half = jsel & 1
    gate = lax.select(sel_s > 0, jnp.int32(0xC080), jnp.int32(0))
    maski = gate << (half * 16)
    poff = pl.multiple_of(tsel * _L, _L)
    pi = pen[pl.ds(poff, _L)]
    pen[pl.ds(poff, _L)] = jnp.where(lane == lsel, pi | maski, pi)
    return maxv, sel


def _greedy_body(nrows, nchunks, vb, xr, tl, outsz, outseq,
                 bufa0, bufa1, bufb0, bufb1, pena, penb, tailba, tailbb,
                 seqba, seqbb, szb, sema0, sema1, semb0, semb1):
    c = lax.axis_index("c")
    s = lax.axis_index("s")
    nc = plsc.get_sparse_core_info().num_cores
    wid = s * nc + c
    lane = lax.iota(jnp.int32, _L)
    nv = nchunks * vb

    szb[...] = jnp.zeros((_L,), jnp.float32)

    def row_body(r, _):
        ba = wid * nrows + r
        bb = wid * nrows + 2 + r

        def pen_init(t, _):
            off = pl.ds(pl.multiple_of(t * _L, _L), _L)
            pena[off] = jnp.zeros((_L,), jnp.int32)
            penb[off] = jnp.zeros((_L,), jnp.int32)
            return 0
        lax.fori_loop(0, _NVREG // 2 + 1, pen_init, 0)

        pltpu.sync_copy(tl.at[pl.ds(ba * nv, nv)], tailba)
        pltpu.sync_copy(tl.at[pl.ds(bb * nv, nv)], tailbb)
        cpa = [pltpu.async_copy(xr.at[ba, pl.ds(0, vb), :], bufa0, sema0)]
        cpb = [pltpu.async_copy(xr.at[bb, pl.ds(0, vb), :], bufb0, semb0)]

        def v_step(bfa, bfb, ci, vl, carry):
            sza, sqa, szbv, sqb = carry
            vglob = ci * vb + vl
            ra, ga = _scan(bfa, pena, tailba, vl, vglob, lane)
            rb, gb = _scan(bfb, penb, tailbb, vl, vglob, lane)
            maxva, sela = _pick(ra, ga, pena, lane)
            maxvb, selb = _pick(rb, gb, penb, lane)

            sqa = jnp.where(lane == (vglob & 15), sela, sqa)
            sqb = jnp.where(lane == (vglob & 15), selb, sqb)

            @pl.when((vglob & 15) == 15)
            def _():
                qoff = pl.ds(pl.multiple_of((vglob >> 4) * _L, _L), _L)
                seqba[qoff] = sqa
                seqbb[qoff] = sqb

            return (sza + maxva, sqa, szbv + maxvb, sqb)

        carry = (jnp.zeros((_L,), jnp.float32), jnp.zeros((_L,), jnp.int32),
                 jnp.zeros((_L,), jnp.float32), jnp.zeros((_L,), jnp.int32))
        for ci in range(nchunks):
            bfa = bufa0 if ci % 2 == 0 else bufa1
            bfb = bufb0 if ci % 2 == 0 else bufb1
            if ci + 1 < nchunks:
                na, sa = (bufa0, sema0) if (ci + 1) % 2 == 0 else (bufa1, sema1)
                nb, sb = (bufb0, semb0) if (ci + 1) % 2 == 0 else (bufb1, semb1)
                cpa.append(pltpu.async_copy(
                    xr.at[ba, pl.ds((ci + 1) * vb, vb), :], na, sa))
                cpb.append(pltpu.async_copy(
                    xr.at[bb, pl.ds((ci + 1) * vb, vb), :], nb, sb))
            cpa[ci].wait()
            cpb[ci].wait()
            carry = lax.fori_loop(
                0, vb, functools.partial(v_step, bfa, bfb, ci), carry)
        sza, _, szbv, _ = carry

        szb[...] = jnp.where(lane == r, -sza, szb[...])
        szb[...] = jnp.where(lane == r + 2, -szbv, szb[...])
        pltpu.sync_copy(seqba, outseq.at[pl.ds(ba * nv, nv)])
        pltpu.sync_copy(seqbb, outseq.at[pl.ds(bb * nv, nv)])
        return 0

    lax.fori_loop(0, nrows // 2, row_body, 0)
    pltpu.sync_copy(szb, outsz.at[pl.ds(wid * _L, _L)])


def kernel(x, u_size, v_size):
    B, V, U1 = x.shape
    info = plsc.get_sparse_core_info()
    nw = info.num_cores * info.num_subcores
    nrows = B // nw
    vb = 16
    nchunks = V // vb

    tl = x[:, :, U1 - 1].reshape(B * V)

    body = functools.partial(_greedy_body, nrows, nchunks, vb)
    run = pl.kernel(
        body,
        mesh=plsc.VectorSubcoreMesh(core_axis_name="c", subcore_axis_name="s"),
        out_type=[
            jax.ShapeDtypeStruct((nw * _L,), jnp.float32),
            jax.ShapeDtypeStruct((B * V,), jnp.int32),
        ],
        scratch_types=[
            pltpu.VMEM((vb, _U1), jnp.float32),
            pltpu.VMEM((vb, _U1), jnp.float32),
            pltpu.VMEM((vb, _U1), jnp.float32),
            pltpu.VMEM((vb, _U1), jnp.float32),
            pltpu.VMEM(((_NVREG // 2 + 1) * _L,), jnp.int32),
            pltpu.VMEM(((_NVREG // 2 + 1) * _L,), jnp.int32),
            pltpu.VMEM((V,), jnp.float32),
            pltpu.VMEM((V,), jnp.float32),
            pltpu.VMEM((V,), jnp.int32),
            pltpu.VMEM((V,), jnp.int32),
            pltpu.VMEM((_L,), jnp.float32),
            pltpu.SemaphoreType.DMA,
            pltpu.SemaphoreType.DMA,
            pltpu.SemaphoreType.DMA,
            pltpu.SemaphoreType.DMA,
        ],
        compiler_params=pltpu.CompilerParams(
            use_tc_tiling_on_sc=True, needs_layout_passes=False),
    )
    outsz, outseq = run(x, tl)
    neg_size = outsz.reshape(nw, _L)[:, :nrows].reshape(B)
    return (neg_size, outseq.reshape(B, V))
